# baseline (speedup 1.0000x reference)
import functools

import jax
import jax.numpy as jnp
from jax import lax
from jax.experimental import pallas as pl
from jax.experimental.pallas import tpu as pltpu
from jax.experimental.pallas import tpu_sc as plsc

B = 4096
F = 256
H = 1024
IPN = 16

NC = 2
NS = 16
NW = NC * NS
HPW = H // NW

BB = 4096


def _build_w_sc(idxt_hbm, w1t_hbm, w_hbm, idx_v, w1_v, w_v):
    wid = lax.axis_index("s") * NC + lax.axis_index("c")
    h0 = wid * HPW

    pltpu.sync_copy(idxt_hbm.at[:, pl.ds((h0 // 128) * 128, 128)], idx_v)
    pltpu.sync_copy(w1t_hbm.at[:, pl.ds((h0 // 128) * 128, 128)], w1_v)
    lh0 = h0 % 128

    zeros = jnp.zeros((16,), jnp.float32)
    lanes = lax.iota(jnp.int32, 16)

    def body(h, carry):
        for c in range(F // 16):
            w_v[h, pl.ds(c * 16, 16)] = zeros
        hvec = jnp.full((16,), h, jnp.int32)
        hslab = jnp.full((16,), lh0 + h, jnp.int32)
        cols = plsc.load_gather(idx_v, [lanes, hslab])
        vals = plsc.load_gather(w1_v, [lanes, hslab])
        plsc.store_scatter(w_v, [hvec, cols], vals)
        return carry

    lax.fori_loop(0, HPW, body, 0)

    pltpu.sync_copy(w_v, w_hbm.at[pl.ds(h0, HPW)])


_sc_build_w = functools.partial(
    pl.kernel,
    out_type=jax.ShapeDtypeStruct((H, F), jnp.float32),
    mesh=plsc.VectorSubcoreMesh(core_axis_name="c", subcore_axis_name="s"),
    compiler_params=pltpu.CompilerParams(
        use_tc_tiling_on_sc=True, needs_layout_passes=False
    ),
    scratch_types=[
        pltpu.VMEM((IPN, 128), jnp.int32),
        pltpu.VMEM((IPN, 128), jnp.float32),
        pltpu.VMEM((HPW, F), jnp.float32),
    ],
)(_build_w_sc)


def _mlp_tc(x_ref, w_ref, b1_ref, w2_ref, b2_ref, out_ref):
    x = x_ref[...]
    w = w_ref[...]
    pre = lax.dot_general(
        x, w,
        (((1,), (1,)), ((), ())),
        preferred_element_type=jnp.float32,
        precision=lax.Precision.DEFAULT,
    )
    hidden = jnp.tanh(pre + b1_ref[...])
    out2 = jnp.sum(hidden * w2_ref[...], axis=1)
    out_ref[...] = jnp.tanh(out2 + b2_ref[0, 0])


def kernel(x, input_indices, w1, b1, w2, b2):
    w_dense = _sc_build_w(input_indices.T, w1.T)

    b1_2d = b1.reshape(1, H)
    b2_2d = b2.reshape(1, 1)

    out = pl.pallas_call(
        _mlp_tc,
        grid=(B // BB,),
        in_specs=[
            pl.BlockSpec((BB, F), lambda i: (i, 0)),
            pl.BlockSpec((H, F), lambda i: (0, 0)),
            pl.BlockSpec((1, H), lambda i: (0, 0)),
            pl.BlockSpec((1, H), lambda i: (0, 0)),
            pl.BlockSpec(memory_space=pltpu.SMEM),
        ],
        out_specs=pl.BlockSpec((BB,), lambda i: (i,)),
        out_shape=jax.ShapeDtypeStruct((B,), jnp.float32),
    )(x, w_dense, b1_2d, w2, b2_2d)

    return out


# speedup vs baseline: 1.0127x; 1.0127x over previous
import functools

import jax
import jax.numpy as jnp
from jax import lax
from jax.experimental import pallas as pl
from jax.experimental.pallas import tpu as pltpu
from jax.experimental.pallas import tpu_sc as plsc

B = 4096
F = 256
H = 1024
IPN = 16

NC = 2
NS = 16
NW = NC * NS
HPW = H // NW

BB = 2048


def _build_w_sc(idxt_hbm, w1t_hbm, w_hbm, idx_v, w1_v, w_v):
    wid = lax.axis_index("s") * NC + lax.axis_index("c")
    h0 = wid * HPW

    pltpu.sync_copy(idxt_hbm.at[:, pl.ds((h0 // 128) * 128, 128)], idx_v)
    pltpu.sync_copy(w1t_hbm.at[:, pl.ds((h0 // 128) * 128, 128)], w1_v)
    lh0 = h0 % 128

    zeros = jnp.zeros((16,), jnp.float32)
    lanes = lax.iota(jnp.int32, 16)

    def body(h, carry):
        for c in range(F // 16):
            w_v[h, pl.ds(c * 16, 16)] = zeros
        hvec = jnp.full((16,), h, jnp.int32)
        hslab = jnp.full((16,), lh0 + h, jnp.int32)
        cols = plsc.load_gather(idx_v, [lanes, hslab])
        vals = plsc.load_gather(w1_v, [lanes, hslab])
        plsc.store_scatter(w_v, [hvec, cols], vals)
        return carry

    lax.fori_loop(0, HPW, body, 0)

    pltpu.sync_copy(w_v, w_hbm.at[pl.ds(h0, HPW)])


_sc_build_w = functools.partial(
    pl.kernel,
    out_type=jax.ShapeDtypeStruct((H, F), jnp.float32),
    mesh=plsc.VectorSubcoreMesh(core_axis_name="c", subcore_axis_name="s"),
    compiler_params=pltpu.CompilerParams(
        use_tc_tiling_on_sc=True, needs_layout_passes=False
    ),
    scratch_types=[
        pltpu.VMEM((IPN, 128), jnp.int32),
        pltpu.VMEM((IPN, 128), jnp.float32),
        pltpu.VMEM((HPW, F), jnp.float32),
    ],
)(_build_w_sc)


def _mlp_tc(x_ref, w_ref, b1_ref, w2_ref, b2_ref, out_ref):
    x = x_ref[...]
    w = w_ref[...]
    pre = lax.dot_general(
        x, w,
        (((1,), (1,)), ((), ())),
        preferred_element_type=jnp.float32,
        precision=lax.Precision.DEFAULT,
    )
    hidden = jnp.tanh(pre + b1_ref[...])
    out2 = jnp.sum(hidden * w2_ref[...], axis=1)
    out_ref[...] = jnp.tanh(out2 + b2_ref[0, 0])


def kernel(x, input_indices, w1, b1, w2, b2):
    w_dense = _sc_build_w(input_indices.T, w1.T)

    b1_2d = b1.reshape(1, H)
    b2_2d = b2.reshape(1, 1)

    out = pl.pallas_call(
        _mlp_tc,
        grid=(B // BB,),
        in_specs=[
            pl.BlockSpec((BB, F), lambda i: (i, 0)),
            pl.BlockSpec((H, F), lambda i: (0, 0)),
            pl.BlockSpec((1, H), lambda i: (0, 0)),
            pl.BlockSpec((1, H), lambda i: (0, 0)),
            pl.BlockSpec(memory_space=pltpu.SMEM),
        ],
        out_specs=pl.BlockSpec((BB,), lambda i: (i,)),
        out_shape=jax.ShapeDtypeStruct((B,), jnp.float32),
    )(x, w_dense, b1_2d, w2, b2_2d)

    return out


# speedup vs baseline: 1.0495x; 1.0364x over previous
import functools

import jax
import jax.numpy as jnp
from jax import lax
from jax.experimental import pallas as pl
from jax.experimental.pallas import tpu as pltpu
from jax.experimental.pallas import tpu_sc as plsc

B = 4096
F = 256
H = 1024
IPN = 16

NC = 2
NS = 16
NW = NC * NS
HPW = H // NW

BB = 512


def _build_w_sc(idxt_hbm, w1t_hbm, w_hbm, idx_v, w1_v, w_v, sem0, sem1):
    wid = lax.axis_index("s") * NC + lax.axis_index("c")
    h0 = wid * HPW

    slab0 = (h0 // 128) * 128
    cp_idx = pltpu.async_copy(idxt_hbm.at[:, pl.ds(slab0, 128)], idx_v, sem0)
    cp_w1 = pltpu.async_copy(w1t_hbm.at[:, pl.ds(slab0, 128)], w1_v, sem1)
    cp_idx.wait()
    cp_w1.wait()
    lh0 = h0 % 128

    zeros = jnp.zeros((16,), jnp.float32)
    lanes = lax.iota(jnp.int32, 16)

    def body(h, carry):
        for c in range(F // 16):
            w_v[h, pl.ds(c * 16, 16)] = zeros
        hvec = jnp.full((16,), h, jnp.int32)
        hslab = jnp.full((16,), lh0 + h, jnp.int32)
        cols = plsc.load_gather(idx_v, [lanes, hslab])
        vals = plsc.load_gather(w1_v, [lanes, hslab])
        plsc.store_scatter(w_v, [hvec, cols], vals)
        return carry

    lax.fori_loop(0, HPW, body, 0)

    pltpu.sync_copy(w_v, w_hbm.at[pl.ds(h0, HPW)])


_sc_build_w = functools.partial(
    pl.kernel,
    out_type=jax.ShapeDtypeStruct((H, F), jnp.float32),
    mesh=plsc.VectorSubcoreMesh(core_axis_name="c", subcore_axis_name="s"),
    compiler_params=pltpu.CompilerParams(
        use_tc_tiling_on_sc=True, needs_layout_passes=False
    ),
    scratch_types=[
        pltpu.VMEM((IPN, 128), jnp.int32),
        pltpu.VMEM((IPN, 128), jnp.float32),
        pltpu.VMEM((HPW, F), jnp.float32),
        pltpu.SemaphoreType.DMA,
        pltpu.SemaphoreType.DMA,
    ],
)(_build_w_sc)


def _mlp_tc(x_ref, w_ref, b1_ref, w2_ref, b2_ref, out_ref):
    x = x_ref[...]
    w = w_ref[...]
    pre = lax.dot_general(
        x, w,
        (((1,), (1,)), ((), ())),
        preferred_element_type=jnp.float32,
        precision=lax.Precision.DEFAULT,
    )
    hidden = jnp.tanh(pre + b1_ref[...])
    out2 = jnp.sum(hidden * w2_ref[...], axis=1)
    out_ref[...] = jnp.tanh(out2 + b2_ref[0, 0])


def kernel(x, input_indices, w1, b1, w2, b2):
    w_dense = _sc_build_w(input_indices.T, w1.T)

    b1_2d = b1.reshape(1, H)
    b2_2d = b2.reshape(1, 1)

    out = pl.pallas_call(
        _mlp_tc,
        grid=(B // BB,),
        in_specs=[
            pl.BlockSpec((BB, F), lambda i: (i, 0)),
            pl.BlockSpec((H, F), lambda i: (0, 0)),
            pl.BlockSpec((1, H), lambda i: (0, 0)),
            pl.BlockSpec((1, H), lambda i: (0, 0)),
            pl.BlockSpec(memory_space=pltpu.SMEM),
        ],
        out_specs=pl.BlockSpec((BB,), lambda i: (i,)),
        out_shape=jax.ShapeDtypeStruct((B,), jnp.float32),
    )(x, w_dense, b1_2d, w2, b2_2d)

    return out


# speedup vs baseline: 1.0497x; 1.0002x over previous
import functools

import jax
import jax.numpy as jnp
from jax import lax
from jax.experimental import pallas as pl
from jax.experimental.pallas import tpu as pltpu
from jax.experimental.pallas import tpu_sc as plsc

B = 4096
F = 256
H = 1024
IPN = 16

NC = 2
NS = 16
NW = NC * NS
HPW = H // NW

BB = 512


def _build_w_sc(idxt_hbm, w1t_hbm, w_hbm, idx_v, w1_v, w_v, sem0, sem1):
    wid = lax.axis_index("s") * NC + lax.axis_index("c")
    h0 = wid * HPW

    slab0 = (h0 // 128) * 128
    cp_idx = pltpu.async_copy(idxt_hbm.at[:, pl.ds(slab0, 128)], idx_v, sem0)
    cp_w1 = pltpu.async_copy(w1t_hbm.at[:, pl.ds(slab0, 128)], w1_v, sem1)
    cp_idx.wait()
    cp_w1.wait()
    lh0 = h0 % 128

    zeros = jnp.zeros((16,), jnp.float32)
    lanes = lax.iota(jnp.int32, 16)

    def body(h, carry):
        for c in range(F // 16):
            w_v[h, pl.ds(c * 16, 16)] = zeros
        hvec = jnp.full((16,), h, jnp.int32)
        hslab = jnp.full((16,), lh0 + h, jnp.int32)
        cols = plsc.load_gather(idx_v, [lanes, hslab])
        vals = plsc.load_gather(w1_v, [lanes, hslab])
        plsc.store_scatter(w_v, [hvec, cols], vals)
        return carry

    lax.fori_loop(0, HPW, body, 0)

    pltpu.sync_copy(w_v, w_hbm.at[pl.ds(h0, HPW)])


_sc_build_w = functools.partial(
    pl.kernel,
    out_type=jax.ShapeDtypeStruct((H, F), jnp.float32),
    mesh=plsc.VectorSubcoreMesh(core_axis_name="c", subcore_axis_name="s"),
    compiler_params=pltpu.CompilerParams(
        use_tc_tiling_on_sc=True, needs_layout_passes=False, skip_device_barrier=True
    ),
    scratch_types=[
        pltpu.VMEM((IPN, 128), jnp.int32),
        pltpu.VMEM((IPN, 128), jnp.float32),
        pltpu.VMEM((HPW, F), jnp.float32),
        pltpu.SemaphoreType.DMA,
        pltpu.SemaphoreType.DMA,
    ],
)(_build_w_sc)


def _mlp_tc(x_ref, w_ref, b1_ref, w2_ref, b2_ref, out_ref):
    x = x_ref[...]
    w = w_ref[...]
    pre = lax.dot_general(
        x, w,
        (((1,), (1,)), ((), ())),
        preferred_element_type=jnp.float32,
        precision=lax.Precision.DEFAULT,
    )
    hidden = jnp.tanh(pre + b1_ref[...])
    out2 = jnp.sum(hidden * w2_ref[...], axis=1)
    out_ref[...] = jnp.tanh(out2 + b2_ref[0, 0])


def kernel(x, input_indices, w1, b1, w2, b2):
    w_dense = _sc_build_w(input_indices.T, w1.T)

    b1_2d = b1.reshape(1, H)
    b2_2d = b2.reshape(1, 1)

    out = pl.pallas_call(
        _mlp_tc,
        grid=(B // BB,),
        in_specs=[
            pl.BlockSpec((BB, F), lambda i: (i, 0)),
            pl.BlockSpec((H, F), lambda i: (0, 0)),
            pl.BlockSpec((1, H), lambda i: (0, 0)),
            pl.BlockSpec((1, H), lambda i: (0, 0)),
            pl.BlockSpec(memory_space=pltpu.SMEM),
        ],
        out_specs=pl.BlockSpec((BB,), lambda i: (i,)),
        out_shape=jax.ShapeDtypeStruct((B,), jnp.float32),
    )(x, w_dense, b1_2d, w2, b2_2d)

    return out
